# baseline (speedup 1.0000x reference)
import functools

import jax
import jax.numpy as jnp
from jax import lax
from jax.experimental import pallas as pl
from jax.experimental.pallas import tpu as pltpu
from jax.experimental.pallas import tpu_sc as plsc

N = 10000
E = 320000
NP = 10240
NTILES = 32
EP = 327680
PB = 8192
NPH = EP // PB
DUMMY = N

_mesh = plsc.VectorSubcoreMesh(core_axis_name="c", subcore_axis_name="s")
_params = pltpu.CompilerParams(needs_layout_passes=False)


def _deg_parts(dst2):
  ept = EP // NTILES

  @functools.partial(
      pl.kernel,
      out_type=jax.ShapeDtypeStruct((NTILES, NP), jnp.float32),
      mesh=_mesh,
      compiler_params=_params,
      scratch_types=[
          pltpu.VMEM((ept,), jnp.int32),
          pltpu.VMEM((NP,), jnp.float32),
      ],
  )
  def k(dst_hbm, out_hbm, dstv, hist):
    c = lax.axis_index("c")
    s = lax.axis_index("s")
    wid = c * 16 + s
    pltpu.sync_copy(dst_hbm.at[wid], dstv)
    zeros = jnp.zeros((16,), jnp.float32)
    ones = jnp.ones((16,), jnp.float32)

    def zbody(i, carry):
      hist[pl.ds(i * 16, 16)] = zeros
      return carry

    lax.fori_loop(0, NP // 16, zbody, 0)

    @plsc.parallel_loop(0, ept // 16, unroll=8)
    def body(i):
      idx = dstv[pl.ds(i * 16, 16)]
      plsc.addupdate_scatter(hist, [idx], ones)
    pltpu.sync_copy(hist, out_hbm.at[wid])

  return k(dst2)


def _edge_scatter_t(gt, src1, dst1):
  d = gt.shape[0]
  cpt = d // NTILES

  @functools.partial(
      pl.kernel,
      out_type=jax.ShapeDtypeStruct((d, NP), jnp.float32),
      mesh=_mesh,
      compiler_params=_params,
      scratch_types=[
          pltpu.VMEM((PB,), jnp.int32),
          pltpu.VMEM((PB,), jnp.int32),
          pltpu.VMEM((PB,), jnp.int32),
          pltpu.VMEM((PB,), jnp.int32),
          pltpu.VMEM((cpt, NP), jnp.float32),
          pltpu.VMEM((cpt, NP), jnp.float32),
          pltpu.SemaphoreType.DMA,
          pltpu.SemaphoreType.DMA,
      ],
  )
  def k(gt_hbm, src_hbm, dst_hbm, out_hbm,
        srcA, dstA, srcB, dstB, slab, acc, semA, semB):
    c = lax.axis_index("c")
    s = lax.axis_index("s")
    r0 = (c * 16 + s) * cpt
    pltpu.sync_copy(gt_hbm.at[pl.ds(r0, cpt)], slab)

    zeros = jnp.zeros((16,), jnp.float32)

    def zbody(i, carry):
      for cc in range(cpt):
        acc[cc, pl.ds(i * 16, 16)] = zeros
      return carry

    lax.fori_loop(0, NP // 16, zbody, 0)

    def start(ph, sv, dv, sem):
      e0 = ph * PB
      pltpu.async_copy(src_hbm.at[pl.ds(e0, PB)], sv, sem)
      pltpu.async_copy(dst_hbm.at[pl.ds(e0, PB)], dv, sem)

    def wait(sv, dv, sem):
      pltpu.make_async_copy(src_hbm.at[pl.ds(0, PB)], sv, sem).wait()
      pltpu.make_async_copy(dst_hbm.at[pl.ds(0, PB)], dv, sem).wait()

    def process(sv, dv):
      cvs = [jnp.full((16,), cc, jnp.int32) for cc in range(cpt)]

      @plsc.parallel_loop(0, PB // 16, unroll=8)
      def ibody(i):
        s16 = sv[pl.ds(i * 16, 16)]
        d16 = dv[pl.ds(i * 16, 16)]
        for cc in range(cpt):
          val = plsc.load_gather(slab, [cvs[cc], s16])
          plsc.addupdate_scatter(acc, [cvs[cc], d16], val)

    start(0, srcA, dstA, semA)
    start(1, srcB, dstB, semB)

    def phases(i, carry):
      phA = 2 * i
      phB = 2 * i + 1
      wait(srcA, dstA, semA)
      process(srcA, dstA)
      start(jnp.minimum(phA + 2, NPH - 2), srcA, dstA, semA)
      wait(srcB, dstB, semB)
      process(srcB, dstB)
      start(jnp.minimum(phB + 2, NPH - 1), srcB, dstB, semB)
      return carry

    lax.fori_loop(0, NPH // 2, phases, 0)
    wait(srcA, dstA, semA)
    wait(srcB, dstB, semB)

    pltpu.sync_copy(acc, out_hbm.at[pl.ds(r0, cpt)])

  return k(gt, src1, dst1)


def _dinv2(deg_parts):
  def body(dp_ref, oc_ref, or_ref):
    deg = jnp.sum(dp_ref[...], axis=0) + 1.0
    dv = lax.rsqrt(deg)
    oc_ref[...] = dv[:, None]
    or_ref[...] = dv[None, :]

  return pl.pallas_call(
      body,
      out_shape=[
          jax.ShapeDtypeStruct((NP, 1), jnp.float32),
          jax.ShapeDtypeStruct((1, NP), jnp.float32),
      ],
  )(deg_parts)


def _tc_scale_matmul_t(xp, w, dinv):
  din, dout = w.shape

  def body(x_ref, w_ref, dv_ref, o_ref):
    h = jnp.dot(x_ref[...], w_ref[...], preferred_element_type=jnp.float32)
    o_ref[...] = jnp.transpose(h * dv_ref[...])

  return pl.pallas_call(
      body,
      grid=(16,),
      in_specs=[
          pl.BlockSpec((640, din), lambda i: (i, 0)),
          pl.BlockSpec((din, dout), lambda i: (0, 0)),
          pl.BlockSpec((640, 1), lambda i: (i, 0)),
      ],
      out_specs=pl.BlockSpec((dout, 640), lambda i: (0, i)),
      out_shape=jax.ShapeDtypeStruct((dout, NP), jnp.float32),
  )(xp, w, dinv)


def _tc_combine_relu_matmul_t(st, gt, dinvr, bc, wt):
  dout, din = wt.shape

  def body(st_ref, gt_ref, dv_ref, b_ref, w_ref, o_ref):
    a = (st_ref[...] + gt_ref[...]) * dv_ref[...] + b_ref[...]
    r = jnp.maximum(a, 0.0)
    h = jnp.dot(w_ref[...], r, preferred_element_type=jnp.float32)
    o_ref[...] = h * dv_ref[...]

  return pl.pallas_call(
      body,
      grid=(16,),
      in_specs=[
          pl.BlockSpec((din, 640), lambda i: (0, i)),
          pl.BlockSpec((din, 640), lambda i: (0, i)),
          pl.BlockSpec((1, 640), lambda i: (0, i)),
          pl.BlockSpec((din, 1), lambda i: (0, 0)),
          pl.BlockSpec((dout, din), lambda i: (0, 0)),
      ],
      out_specs=pl.BlockSpec((dout, 640), lambda i: (0, i)),
      out_shape=jax.ShapeDtypeStruct((dout, NP), jnp.float32),
  )(st, gt, dinvr, bc, wt)


def _tc_combine_logsoftmax_t(st, gt, dinvr, bc):
  d = gt.shape[0]

  def body(st_ref, gt_ref, dv_ref, b_ref, o_ref):
    o = (st_ref[...] + gt_ref[...]) * dv_ref[...] + b_ref[...]
    m = jnp.max(o, axis=0, keepdims=True)
    e = jnp.exp(o - m)
    lse = jnp.log(jnp.sum(e, axis=0, keepdims=True))
    o_ref[...] = jnp.transpose(o - m - lse)

  return pl.pallas_call(
      body,
      grid=(16,),
      in_specs=[
          pl.BlockSpec((d, 640), lambda i: (0, i)),
          pl.BlockSpec((d, 640), lambda i: (0, i)),
          pl.BlockSpec((1, 640), lambda i: (0, i)),
          pl.BlockSpec((d, 1), lambda i: (0, 0)),
      ],
      out_specs=pl.BlockSpec((640, d), lambda i: (i, 0)),
      out_shape=jax.ShapeDtypeStruct((NP, d), jnp.float32),
  )(st, gt, dinvr, bc)


def kernel(x, edge_index, W1, b1, W2, b2):
  src = edge_index[0].astype(jnp.int32)
  dst = edge_index[1].astype(jnp.int32)
  pad = EP - E
  src1 = jnp.concatenate([src, jnp.zeros((pad,), jnp.int32)])
  dst1 = jnp.concatenate([dst, jnp.full((pad,), DUMMY, jnp.int32)])
  dsth = dst1.reshape(NTILES, EP // NTILES)

  dparts = _deg_parts(dsth)
  dinvc, dinvr = _dinv2(dparts)

  xp = jnp.pad(x, ((0, NP - N), (0, 0)))
  g1t = _tc_scale_matmul_t(xp, W1, dinvc)
  s1t = _edge_scatter_t(g1t, src1, dst1)
  g2t = _tc_combine_relu_matmul_t(s1t, g1t, dinvr, b1.reshape(-1, 1), W2.T)
  s2t = _edge_scatter_t(g2t, src1, dst1)
  out = _tc_combine_logsoftmax_t(s2t, g2t, dinvr, b2.reshape(-1, 1))
  return out[:N]


# speedup vs baseline: 1.0062x; 1.0062x over previous
import functools

import jax
import jax.numpy as jnp
from jax import lax
from jax.experimental import pallas as pl
from jax.experimental.pallas import tpu as pltpu
from jax.experimental.pallas import tpu_sc as plsc

N = 10000
E = 320000
NP = 10240
NTILES = 32
EP = 327680
PB = 8192
NPH = EP // PB
DUMMY = N

_mesh = plsc.VectorSubcoreMesh(core_axis_name="c", subcore_axis_name="s")
_params = pltpu.CompilerParams(needs_layout_passes=False)


def _deg_parts(dst2):
  ept = EP // NTILES

  @functools.partial(
      pl.kernel,
      out_type=jax.ShapeDtypeStruct((NTILES, NP), jnp.float32),
      mesh=_mesh,
      compiler_params=_params,
      scratch_types=[
          pltpu.VMEM((ept,), jnp.int32),
          pltpu.VMEM((NP,), jnp.float32),
      ],
  )
  def k(dst_hbm, out_hbm, dstv, hist):
    c = lax.axis_index("c")
    s = lax.axis_index("s")
    wid = c * 16 + s
    pltpu.sync_copy(dst_hbm.at[wid], dstv)
    zeros = jnp.zeros((16,), jnp.float32)
    ones = jnp.ones((16,), jnp.float32)

    def zbody(i, carry):
      hist[pl.ds(i * 16, 16)] = zeros
      return carry

    lax.fori_loop(0, NP // 16, zbody, 0)

    @plsc.parallel_loop(0, ept // 16, unroll=8)
    def body(i):
      idx = dstv[pl.ds(i * 16, 16)]
      plsc.addupdate_scatter(hist, [idx], ones)
    pltpu.sync_copy(hist, out_hbm.at[wid])

  return k(dst2)


def _edge_scatter_t(gt, src1, dst1):
  d = gt.shape[0]
  cpt = d // NTILES

  @functools.partial(
      pl.kernel,
      out_type=jax.ShapeDtypeStruct((d, NP), jnp.float32),
      mesh=_mesh,
      compiler_params=_params,
      scratch_types=[
          pltpu.VMEM((PB,), jnp.int32),
          pltpu.VMEM((PB,), jnp.int32),
          pltpu.VMEM((PB,), jnp.int32),
          pltpu.VMEM((PB,), jnp.int32),
          pltpu.VMEM((cpt, NP), jnp.float32),
          pltpu.VMEM((cpt, NP), jnp.float32),
          pltpu.SemaphoreType.DMA,
          pltpu.SemaphoreType.DMA,
      ],
  )
  def k(gt_hbm, src_hbm, dst_hbm, out_hbm,
        srcA, dstA, srcB, dstB, slab, acc, semA, semB):
    c = lax.axis_index("c")
    s = lax.axis_index("s")
    r0 = (c * 16 + s) * cpt
    pltpu.sync_copy(gt_hbm.at[pl.ds(r0, cpt)], slab)

    zeros = jnp.zeros((16,), jnp.float32)

    def zbody(i, carry):
      for cc in range(cpt):
        acc[cc, pl.ds(i * 16, 16)] = zeros
      return carry

    lax.fori_loop(0, NP // 16, zbody, 0)

    def start(ph, sv, dv, sem):
      e0 = ph * PB
      pltpu.async_copy(src_hbm.at[pl.ds(e0, PB)], sv, sem)
      pltpu.async_copy(dst_hbm.at[pl.ds(e0, PB)], dv, sem)

    def wait(sv, dv, sem):
      pltpu.make_async_copy(src_hbm.at[pl.ds(0, PB)], sv, sem).wait()
      pltpu.make_async_copy(dst_hbm.at[pl.ds(0, PB)], dv, sem).wait()

    def process(sv, dv):
      cvs = [jnp.full((16,), cc, jnp.int32) for cc in range(cpt)]

      @plsc.parallel_loop(0, PB // 16, unroll=4)
      def ibody(i):
        s16 = sv[pl.ds(i * 16, 16)]
        d16 = dv[pl.ds(i * 16, 16)]
        for cc in range(cpt):
          val = plsc.load_gather(slab, [cvs[cc], s16])
          plsc.addupdate_scatter(acc, [cvs[cc], d16], val)

    start(0, srcA, dstA, semA)
    start(1, srcB, dstB, semB)

    def phases(i, carry):
      phA = 2 * i
      phB = 2 * i + 1
      wait(srcA, dstA, semA)
      process(srcA, dstA)
      start(jnp.minimum(phA + 2, NPH - 2), srcA, dstA, semA)
      wait(srcB, dstB, semB)
      process(srcB, dstB)
      start(jnp.minimum(phB + 2, NPH - 1), srcB, dstB, semB)
      return carry

    lax.fori_loop(0, NPH // 2, phases, 0)
    wait(srcA, dstA, semA)
    wait(srcB, dstB, semB)

    pltpu.sync_copy(acc, out_hbm.at[pl.ds(r0, cpt)])

  return k(gt, src1, dst1)


def _dinv2(deg_parts):
  def body(dp_ref, oc_ref, or_ref):
    deg = jnp.sum(dp_ref[...], axis=0) + 1.0
    dv = lax.rsqrt(deg)
    oc_ref[...] = dv[:, None]
    or_ref[...] = dv[None, :]

  return pl.pallas_call(
      body,
      out_shape=[
          jax.ShapeDtypeStruct((NP, 1), jnp.float32),
          jax.ShapeDtypeStruct((1, NP), jnp.float32),
      ],
  )(deg_parts)


def _tc_scale_matmul_t(xp, w, dinv):
  din, dout = w.shape

  def body(x_ref, w_ref, dv_ref, o_ref):
    h = jnp.dot(x_ref[...], w_ref[...], preferred_element_type=jnp.float32)
    o_ref[...] = jnp.transpose(h * dv_ref[...])

  return pl.pallas_call(
      body,
      grid=(16,),
      in_specs=[
          pl.BlockSpec((640, din), lambda i: (i, 0)),
          pl.BlockSpec((din, dout), lambda i: (0, 0)),
          pl.BlockSpec((640, 1), lambda i: (i, 0)),
      ],
      out_specs=pl.BlockSpec((dout, 640), lambda i: (0, i)),
      out_shape=jax.ShapeDtypeStruct((dout, NP), jnp.float32),
  )(xp, w, dinv)


def _tc_combine_relu_matmul_t(st, gt, dinvr, bc, wt):
  dout, din = wt.shape

  def body(st_ref, gt_ref, dv_ref, b_ref, w_ref, o_ref):
    a = (st_ref[...] + gt_ref[...]) * dv_ref[...] + b_ref[...]
    r = jnp.maximum(a, 0.0)
    h = jnp.dot(w_ref[...], r, preferred_element_type=jnp.float32)
    o_ref[...] = h * dv_ref[...]

  return pl.pallas_call(
      body,
      grid=(16,),
      in_specs=[
          pl.BlockSpec((din, 640), lambda i: (0, i)),
          pl.BlockSpec((din, 640), lambda i: (0, i)),
          pl.BlockSpec((1, 640), lambda i: (0, i)),
          pl.BlockSpec((din, 1), lambda i: (0, 0)),
          pl.BlockSpec((dout, din), lambda i: (0, 0)),
      ],
      out_specs=pl.BlockSpec((dout, 640), lambda i: (0, i)),
      out_shape=jax.ShapeDtypeStruct((dout, NP), jnp.float32),
  )(st, gt, dinvr, bc, wt)


def _tc_combine_logsoftmax_t(st, gt, dinvr, bc):
  d = gt.shape[0]

  def body(st_ref, gt_ref, dv_ref, b_ref, o_ref):
    o = (st_ref[...] + gt_ref[...]) * dv_ref[...] + b_ref[...]
    m = jnp.max(o, axis=0, keepdims=True)
    e = jnp.exp(o - m)
    lse = jnp.log(jnp.sum(e, axis=0, keepdims=True))
    o_ref[...] = jnp.transpose(o - m - lse)

  return pl.pallas_call(
      body,
      grid=(16,),
      in_specs=[
          pl.BlockSpec((d, 640), lambda i: (0, i)),
          pl.BlockSpec((d, 640), lambda i: (0, i)),
          pl.BlockSpec((1, 640), lambda i: (0, i)),
          pl.BlockSpec((d, 1), lambda i: (0, 0)),
      ],
      out_specs=pl.BlockSpec((640, d), lambda i: (i, 0)),
      out_shape=jax.ShapeDtypeStruct((NP, d), jnp.float32),
  )(st, gt, dinvr, bc)


def kernel(x, edge_index, W1, b1, W2, b2):
  src = edge_index[0].astype(jnp.int32)
  dst = edge_index[1].astype(jnp.int32)
  pad = EP - E
  src1 = jnp.concatenate([src, jnp.zeros((pad,), jnp.int32)])
  dst1 = jnp.concatenate([dst, jnp.full((pad,), DUMMY, jnp.int32)])
  dsth = dst1.reshape(NTILES, EP // NTILES)

  dparts = _deg_parts(dsth)
  dinvc, dinvr = _dinv2(dparts)

  xp = jnp.pad(x, ((0, NP - N), (0, 0)))
  g1t = _tc_scale_matmul_t(xp, W1, dinvc)
  s1t = _edge_scatter_t(g1t, src1, dst1)
  g2t = _tc_combine_relu_matmul_t(s1t, g1t, dinvr, b1.reshape(-1, 1), W2.T)
  s2t = _edge_scatter_t(g2t, src1, dst1)
  out = _tc_combine_logsoftmax_t(s2t, g2t, dinvr, b2.reshape(-1, 1))
  return out[:N]


# speedup vs baseline: 1.0182x; 1.0119x over previous
import functools

import jax
import jax.numpy as jnp
from jax import lax
from jax.experimental import pallas as pl
from jax.experimental.pallas import tpu as pltpu
from jax.experimental.pallas import tpu_sc as plsc

N = 10000
E = 320000
NP = 10240
NTILES = 32
EP = 327680
PB = 5120
NPH = EP // PB
DUMMY = N

_mesh = plsc.VectorSubcoreMesh(core_axis_name="c", subcore_axis_name="s")
_params = pltpu.CompilerParams(needs_layout_passes=False)


def _deg_parts(dst2):
  ept = EP // NTILES

  @functools.partial(
      pl.kernel,
      out_type=jax.ShapeDtypeStruct((NTILES, NP), jnp.float32),
      mesh=_mesh,
      compiler_params=_params,
      scratch_types=[
          pltpu.VMEM((ept,), jnp.int32),
          pltpu.VMEM((NP,), jnp.float32),
      ],
  )
  def k(dst_hbm, out_hbm, dstv, hist):
    c = lax.axis_index("c")
    s = lax.axis_index("s")
    wid = c * 16 + s
    pltpu.sync_copy(dst_hbm.at[wid], dstv)
    zeros = jnp.zeros((16,), jnp.float32)
    ones = jnp.ones((16,), jnp.float32)

    def zbody(i, carry):
      hist[pl.ds(i * 16, 16)] = zeros
      return carry

    lax.fori_loop(0, NP // 16, zbody, 0)

    @plsc.parallel_loop(0, ept // 16, unroll=8)
    def body(i):
      idx = dstv[pl.ds(i * 16, 16)]
      plsc.addupdate_scatter(hist, [idx], ones)
    pltpu.sync_copy(hist, out_hbm.at[wid])

  return k(dst2)


def _edge_scatter_t(gt, src1, dst1):
  d = gt.shape[0]
  cpt = d // NTILES

  @functools.partial(
      pl.kernel,
      out_type=jax.ShapeDtypeStruct((d, NP), jnp.float32),
      mesh=_mesh,
      compiler_params=_params,
      scratch_types=[
          pltpu.VMEM((PB,), jnp.int32),
          pltpu.VMEM((PB,), jnp.int32),
          pltpu.VMEM((PB,), jnp.int32),
          pltpu.VMEM((PB,), jnp.int32),
          pltpu.VMEM((cpt, NP), jnp.float32),
          pltpu.VMEM((cpt, NP), jnp.float32),
          pltpu.SemaphoreType.DMA,
          pltpu.SemaphoreType.DMA,
      ],
  )
  def k(gt_hbm, src_hbm, dst_hbm, out_hbm,
        srcA, dstA, srcB, dstB, slab, acc, semA, semB):
    c = lax.axis_index("c")
    s = lax.axis_index("s")
    r0 = (c * 16 + s) * cpt
    pltpu.sync_copy(gt_hbm.at[pl.ds(r0, cpt)], slab)

    zeros = jnp.zeros((16,), jnp.float32)

    def zbody(i, carry):
      for cc in range(cpt):
        acc[cc, pl.ds(i * 16, 16)] = zeros
      return carry

    lax.fori_loop(0, NP // 16, zbody, 0)

    def start(ph, sv, dv, sem):
      e0 = ph * PB
      pltpu.async_copy(src_hbm.at[pl.ds(e0, PB)], sv, sem)
      pltpu.async_copy(dst_hbm.at[pl.ds(e0, PB)], dv, sem)

    def wait(sv, dv, sem):
      pltpu.make_async_copy(src_hbm.at[pl.ds(0, PB)], sv, sem).wait()
      pltpu.make_async_copy(dst_hbm.at[pl.ds(0, PB)], dv, sem).wait()

    def process(sv, dv):
      cvs = [jnp.full((16,), cc, jnp.int32) for cc in range(cpt)]

      @plsc.parallel_loop(0, PB // 16, unroll=4)
      def ibody(i):
        s16 = sv[pl.ds(i * 16, 16)]
        d16 = dv[pl.ds(i * 16, 16)]
        for cc in range(cpt):
          val = plsc.load_gather(slab, [cvs[cc], s16])
          plsc.addupdate_scatter(acc, [cvs[cc], d16], val)

    start(0, srcA, dstA, semA)
    start(1, srcB, dstB, semB)

    def phases(i, carry):
      phA = 2 * i
      phB = 2 * i + 1
      wait(srcA, dstA, semA)
      process(srcA, dstA)
      start(jnp.minimum(phA + 2, NPH - 2), srcA, dstA, semA)
      wait(srcB, dstB, semB)
      process(srcB, dstB)
      start(jnp.minimum(phB + 2, NPH - 1), srcB, dstB, semB)
      return carry

    lax.fori_loop(0, NPH // 2, phases, 0)
    wait(srcA, dstA, semA)
    wait(srcB, dstB, semB)

    pltpu.sync_copy(acc, out_hbm.at[pl.ds(r0, cpt)])

  return k(gt, src1, dst1)


def _dinv2(deg_parts):
  def body(dp_ref, oc_ref, or_ref):
    deg = jnp.sum(dp_ref[...], axis=0) + 1.0
    dv = lax.rsqrt(deg)
    oc_ref[...] = dv[:, None]
    or_ref[...] = dv[None, :]

  return pl.pallas_call(
      body,
      out_shape=[
          jax.ShapeDtypeStruct((NP, 1), jnp.float32),
          jax.ShapeDtypeStruct((1, NP), jnp.float32),
      ],
  )(deg_parts)


def _tc_scale_matmul_t(xp, w, dinv):
  din, dout = w.shape

  def body(x_ref, w_ref, dv_ref, o_ref):
    h = jnp.dot(x_ref[...], w_ref[...], preferred_element_type=jnp.float32)
    o_ref[...] = jnp.transpose(h * dv_ref[...])

  return pl.pallas_call(
      body,
      grid=(16,),
      in_specs=[
          pl.BlockSpec((640, din), lambda i: (i, 0)),
          pl.BlockSpec((din, dout), lambda i: (0, 0)),
          pl.BlockSpec((640, 1), lambda i: (i, 0)),
      ],
      out_specs=pl.BlockSpec((dout, 640), lambda i: (0, i)),
      out_shape=jax.ShapeDtypeStruct((dout, NP), jnp.float32),
  )(xp, w, dinv)


def _tc_combine_relu_matmul_t(st, gt, dinvr, bc, wt):
  dout, din = wt.shape

  def body(st_ref, gt_ref, dv_ref, b_ref, w_ref, o_ref):
    a = (st_ref[...] + gt_ref[...]) * dv_ref[...] + b_ref[...]
    r = jnp.maximum(a, 0.0)
    h = jnp.dot(w_ref[...], r, preferred_element_type=jnp.float32)
    o_ref[...] = h * dv_ref[...]

  return pl.pallas_call(
      body,
      grid=(16,),
      in_specs=[
          pl.BlockSpec((din, 640), lambda i: (0, i)),
          pl.BlockSpec((din, 640), lambda i: (0, i)),
          pl.BlockSpec((1, 640), lambda i: (0, i)),
          pl.BlockSpec((din, 1), lambda i: (0, 0)),
          pl.BlockSpec((dout, din), lambda i: (0, 0)),
      ],
      out_specs=pl.BlockSpec((dout, 640), lambda i: (0, i)),
      out_shape=jax.ShapeDtypeStruct((dout, NP), jnp.float32),
  )(st, gt, dinvr, bc, wt)


def _tc_combine_logsoftmax_t(st, gt, dinvr, bc):
  d = gt.shape[0]

  def body(st_ref, gt_ref, dv_ref, b_ref, o_ref):
    o = (st_ref[...] + gt_ref[...]) * dv_ref[...] + b_ref[...]
    m = jnp.max(o, axis=0, keepdims=True)
    e = jnp.exp(o - m)
    lse = jnp.log(jnp.sum(e, axis=0, keepdims=True))
    o_ref[...] = jnp.transpose(o - m - lse)

  return pl.pallas_call(
      body,
      grid=(16,),
      in_specs=[
          pl.BlockSpec((d, 640), lambda i: (0, i)),
          pl.BlockSpec((d, 640), lambda i: (0, i)),
          pl.BlockSpec((1, 640), lambda i: (0, i)),
          pl.BlockSpec((d, 1), lambda i: (0, 0)),
      ],
      out_specs=pl.BlockSpec((640, d), lambda i: (i, 0)),
      out_shape=jax.ShapeDtypeStruct((NP, d), jnp.float32),
  )(st, gt, dinvr, bc)


def kernel(x, edge_index, W1, b1, W2, b2):
  src = edge_index[0].astype(jnp.int32)
  dst = edge_index[1].astype(jnp.int32)
  pad = EP - E
  src1 = jnp.concatenate([src, jnp.zeros((pad,), jnp.int32)])
  dst1 = jnp.concatenate([dst, jnp.full((pad,), DUMMY, jnp.int32)])
  dsth = dst1.reshape(NTILES, EP // NTILES)

  dparts = _deg_parts(dsth)
  dinvc, dinvr = _dinv2(dparts)

  xp = jnp.pad(x, ((0, NP - N), (0, 0)))
  g1t = _tc_scale_matmul_t(xp, W1, dinvc)
  s1t = _edge_scatter_t(g1t, src1, dst1)
  g2t = _tc_combine_relu_matmul_t(s1t, g1t, dinvr, b1.reshape(-1, 1), W2.T)
  s2t = _edge_scatter_t(g2t, src1, dst1)
  out = _tc_combine_logsoftmax_t(s2t, g2t, dinvr, b2.reshape(-1, 1))
  return out[:N]


# speedup vs baseline: 1.0860x; 1.0666x over previous
import functools

import jax
import jax.numpy as jnp
from jax import lax
from jax.experimental import pallas as pl
from jax.experimental.pallas import tpu as pltpu
from jax.experimental.pallas import tpu_sc as plsc

N = 10000
E = 320000
NP = 10240
NTILES = 32
EP = 327680
PB = 5120
NPH = EP // PB
DUMMY = N

_mesh = plsc.VectorSubcoreMesh(core_axis_name="c", subcore_axis_name="s")
_params = pltpu.CompilerParams(needs_layout_passes=False)


def _deg_parts(dst2):
  ept = EP // NTILES

  @functools.partial(
      pl.kernel,
      out_type=jax.ShapeDtypeStruct((NTILES, NP), jnp.float32),
      mesh=_mesh,
      compiler_params=_params,
      scratch_types=[
          pltpu.VMEM((ept,), jnp.int32),
          pltpu.VMEM((NP,), jnp.float32),
      ],
  )
  def k(pk_hbm, out_hbm, pkv, hist):
    c = lax.axis_index("c")
    s = lax.axis_index("s")
    wid = c * 16 + s
    pltpu.sync_copy(pk_hbm.at[wid], pkv)
    zeros = jnp.zeros((16,), jnp.float32)
    ones = jnp.ones((16,), jnp.float32)

    def zbody(i, carry):
      hist[pl.ds(i * 16, 16)] = zeros
      return carry

    lax.fori_loop(0, NP // 16, zbody, 0)

    @plsc.parallel_loop(0, ept // 16, unroll=8)
    def body(i):
      idx = lax.shift_right_logical(pkv[pl.ds(i * 16, 16)], 16)
      plsc.addupdate_scatter(hist, [idx], ones)
    pltpu.sync_copy(hist, out_hbm.at[wid])

  return k(dst2)


def _edge_scatter_t(gt, pk1):
  d = gt.shape[0]
  cpt = d // NTILES

  @functools.partial(
      pl.kernel,
      out_type=jax.ShapeDtypeStruct((d, NP), jnp.float32),
      mesh=_mesh,
      compiler_params=_params,
      scratch_types=[
          pltpu.VMEM((PB,), jnp.int32),
          pltpu.VMEM((PB,), jnp.int32),
          pltpu.VMEM((cpt, NP), jnp.float32),
          pltpu.VMEM((cpt, NP), jnp.float32),
          pltpu.SemaphoreType.DMA,
          pltpu.SemaphoreType.DMA,
      ],
  )
  def k(gt_hbm, pk_hbm, out_hbm, pkA, pkB, slab, acc, semA, semB):
    c = lax.axis_index("c")
    s = lax.axis_index("s")
    r0 = (c * 16 + s) * cpt
    pltpu.sync_copy(gt_hbm.at[pl.ds(r0, cpt)], slab)

    zeros = jnp.zeros((16,), jnp.float32)

    def zbody(i, carry):
      for cc in range(cpt):
        acc[cc, pl.ds(i * 16, 16)] = zeros
      return carry

    lax.fori_loop(0, NP // 16, zbody, 0)

    def start(ph, pv, sem):
      pltpu.async_copy(pk_hbm.at[pl.ds(ph * PB, PB)], pv, sem)

    def wait(pv, sem):
      pltpu.make_async_copy(pk_hbm.at[pl.ds(0, PB)], pv, sem).wait()

    mask = jnp.full((16,), 0xFFFF, jnp.int32)

    def process(pv):
      cvs = [jnp.full((16,), cc, jnp.int32) for cc in range(cpt)]

      @plsc.parallel_loop(0, PB // 16, unroll=4)
      def ibody(i):
        pk16 = pv[pl.ds(i * 16, 16)]
        s16 = jnp.bitwise_and(pk16, mask)
        d16 = lax.shift_right_logical(pk16, 16)
        for cc in range(cpt):
          val = plsc.load_gather(slab, [cvs[cc], s16])
          plsc.addupdate_scatter(acc, [cvs[cc], d16], val)

    start(0, pkA, semA)
    start(1, pkB, semB)

    def phases(i, carry):
      phA = 2 * i
      phB = 2 * i + 1
      wait(pkA, semA)
      process(pkA)
      start(jnp.minimum(phA + 2, NPH - 2), pkA, semA)
      wait(pkB, semB)
      process(pkB)
      start(jnp.minimum(phB + 2, NPH - 1), pkB, semB)
      return carry

    lax.fori_loop(0, NPH // 2, phases, 0)
    wait(pkA, semA)
    wait(pkB, semB)

    pltpu.sync_copy(acc, out_hbm.at[pl.ds(r0, cpt)])

  return k(gt, pk1)


def _dinv2(deg_parts):
  def body(dp_ref, oc_ref, or_ref):
    deg = jnp.sum(dp_ref[...], axis=0) + 1.0
    dv = lax.rsqrt(deg)
    oc_ref[...] = dv[:, None]
    or_ref[...] = dv[None, :]

  return pl.pallas_call(
      body,
      out_shape=[
          jax.ShapeDtypeStruct((NP, 1), jnp.float32),
          jax.ShapeDtypeStruct((1, NP), jnp.float32),
      ],
  )(deg_parts)


def _tc_scale_matmul_t(xp, w, dinv):
  din, dout = w.shape

  def body(x_ref, w_ref, dv_ref, o_ref):
    h = jnp.dot(x_ref[...], w_ref[...], preferred_element_type=jnp.float32)
    o_ref[...] = jnp.transpose(h * dv_ref[...])

  return pl.pallas_call(
      body,
      grid=(16,),
      in_specs=[
          pl.BlockSpec((640, din), lambda i: (i, 0)),
          pl.BlockSpec((din, dout), lambda i: (0, 0)),
          pl.BlockSpec((640, 1), lambda i: (i, 0)),
      ],
      out_specs=pl.BlockSpec((dout, 640), lambda i: (0, i)),
      out_shape=jax.ShapeDtypeStruct((dout, NP), jnp.float32),
  )(xp, w, dinv)


def _tc_combine_relu_matmul_t(st, gt, dinvr, bc, wt):
  dout, din = wt.shape

  def body(st_ref, gt_ref, dv_ref, b_ref, w_ref, o_ref):
    a = (st_ref[...] + gt_ref[...]) * dv_ref[...] + b_ref[...]
    r = jnp.maximum(a, 0.0)
    h = jnp.dot(w_ref[...], r, preferred_element_type=jnp.float32)
    o_ref[...] = h * dv_ref[...]

  return pl.pallas_call(
      body,
      grid=(16,),
      in_specs=[
          pl.BlockSpec((din, 640), lambda i: (0, i)),
          pl.BlockSpec((din, 640), lambda i: (0, i)),
          pl.BlockSpec((1, 640), lambda i: (0, i)),
          pl.BlockSpec((din, 1), lambda i: (0, 0)),
          pl.BlockSpec((dout, din), lambda i: (0, 0)),
      ],
      out_specs=pl.BlockSpec((dout, 640), lambda i: (0, i)),
      out_shape=jax.ShapeDtypeStruct((dout, NP), jnp.float32),
  )(st, gt, dinvr, bc, wt)


def _tc_combine_logsoftmax_t(st, gt, dinvr, bc):
  d = gt.shape[0]

  def body(st_ref, gt_ref, dv_ref, b_ref, o_ref):
    o = (st_ref[...] + gt_ref[...]) * dv_ref[...] + b_ref[...]
    m = jnp.max(o, axis=0, keepdims=True)
    e = jnp.exp(o - m)
    lse = jnp.log(jnp.sum(e, axis=0, keepdims=True))
    o_ref[...] = jnp.transpose(o - m - lse)

  return pl.pallas_call(
      body,
      grid=(16,),
      in_specs=[
          pl.BlockSpec((d, 640), lambda i: (0, i)),
          pl.BlockSpec((d, 640), lambda i: (0, i)),
          pl.BlockSpec((1, 640), lambda i: (0, i)),
          pl.BlockSpec((d, 1), lambda i: (0, 0)),
      ],
      out_specs=pl.BlockSpec((640, d), lambda i: (i, 0)),
      out_shape=jax.ShapeDtypeStruct((N, d), jnp.float32),
  )(st, gt, dinvr, bc)


def kernel(x, edge_index, W1, b1, W2, b2):
  src = edge_index[0].astype(jnp.int32)
  dst = edge_index[1].astype(jnp.int32)
  pad = EP - E
  pk = jnp.bitwise_or(src, jnp.left_shift(dst, 16))
  pk1 = jnp.concatenate([pk, jnp.full((pad,), DUMMY << 16, jnp.int32)])
  pkh = pk1.reshape(NTILES, EP // NTILES)

  dparts = _deg_parts(pkh)
  dinvc, dinvr = _dinv2(dparts)

  g1t = _tc_scale_matmul_t(x, W1, dinvc)
  s1t = _edge_scatter_t(g1t, pk1)
  g2t = _tc_combine_relu_matmul_t(s1t, g1t, dinvr, b1.reshape(-1, 1), W2.T)
  s2t = _edge_scatter_t(g2t, pk1)
  return _tc_combine_logsoftmax_t(s2t, g2t, dinvr, b2.reshape(-1, 1))


# speedup vs baseline: 1.1193x; 1.0307x over previous
import functools

import jax
import jax.numpy as jnp
from jax import lax
from jax.experimental import pallas as pl
from jax.experimental.pallas import tpu as pltpu
from jax.experimental.pallas import tpu_sc as plsc

N = 10000
E = 320000
NP = 10240
NTILES = 32
EP = 327680
PB = 5120
NPH = EP // PB
DUMMY = N

_mesh = plsc.VectorSubcoreMesh(core_axis_name="c", subcore_axis_name="s")
_params = pltpu.CompilerParams(needs_layout_passes=False)


def _deg_parts(dst2):
  ept = EP // NTILES

  @functools.partial(
      pl.kernel,
      out_type=jax.ShapeDtypeStruct((NTILES, NP), jnp.float32),
      mesh=_mesh,
      compiler_params=_params,
      scratch_types=[
          pltpu.VMEM((ept,), jnp.int32),
          pltpu.VMEM((NP,), jnp.float32),
      ],
  )
  def k(pk_hbm, out_hbm, pkv, hist):
    c = lax.axis_index("c")
    s = lax.axis_index("s")
    wid = c * 16 + s
    pltpu.sync_copy(pk_hbm.at[wid], pkv)
    zeros = jnp.zeros((16,), jnp.float32)
    ones = jnp.ones((16,), jnp.float32)

    def zbody(i, carry):
      hist[pl.ds(i * 16, 16)] = zeros
      return carry

    lax.fori_loop(0, NP // 16, zbody, 0)

    @plsc.parallel_loop(0, ept // 16, unroll=8)
    def body(i):
      idx = lax.shift_right_logical(pkv[pl.ds(i * 16, 16)], 16)
      plsc.addupdate_scatter(hist, [idx], ones)
    pltpu.sync_copy(hist, out_hbm.at[wid])

  return k(dst2)


def _edge_scatter_t(gt, pk1):
  d = gt.shape[0]
  cpt = d // NTILES

  @functools.partial(
      pl.kernel,
      out_type=jax.ShapeDtypeStruct((d, NP), jnp.float32),
      mesh=_mesh,
      compiler_params=_params,
      scratch_types=[
          pltpu.VMEM((PB,), jnp.int32),
          pltpu.VMEM((PB,), jnp.int32),
          pltpu.VMEM((cpt, NP), jnp.float32),
          pltpu.VMEM((cpt, NP), jnp.float32),
          pltpu.SemaphoreType.DMA,
          pltpu.SemaphoreType.DMA,
      ],
  )
  def k(gt_hbm, pk_hbm, out_hbm, pkA, pkB, slab, acc, semA, semB):
    c = lax.axis_index("c")
    s = lax.axis_index("s")
    r0 = (c * 16 + s) * cpt
    pltpu.sync_copy(gt_hbm.at[pl.ds(r0, cpt)], slab)

    zeros = jnp.zeros((16,), jnp.float32)

    def zbody(i, carry):
      for cc in range(cpt):
        acc[cc, pl.ds(i * 16, 16)] = zeros
      return carry

    lax.fori_loop(0, NP // 16, zbody, 0)

    def start(ph, pv, sem):
      pltpu.async_copy(pk_hbm.at[pl.ds(ph * PB, PB)], pv, sem)

    def wait(pv, sem):
      pltpu.make_async_copy(pk_hbm.at[pl.ds(0, PB)], pv, sem).wait()

    mask = jnp.full((16,), 0xFFFF, jnp.int32)

    def process(pv):
      cvs = [jnp.full((16,), cc, jnp.int32) for cc in range(cpt)]

      @plsc.parallel_loop(0, PB // 16, unroll=4)
      def ibody(i):
        pk16 = pv[pl.ds(i * 16, 16)]
        s16 = jnp.bitwise_and(pk16, mask)
        d16 = lax.shift_right_logical(pk16, 16)
        for cc in range(cpt):
          val = plsc.load_gather(slab, [cvs[cc], s16])
          plsc.addupdate_scatter(acc, [cvs[cc], d16], val)

    start(0, pkA, semA)
    start(1, pkB, semB)

    def phases(i, carry):
      phA = 2 * i
      phB = 2 * i + 1
      wait(pkA, semA)
      process(pkA)
      start(jnp.minimum(phA + 2, NPH - 2), pkA, semA)
      wait(pkB, semB)
      process(pkB)
      start(jnp.minimum(phB + 2, NPH - 1), pkB, semB)
      return carry

    lax.fori_loop(0, NPH // 2, phases, 0)
    wait(pkA, semA)
    wait(pkB, semB)

    pltpu.sync_copy(acc, out_hbm.at[pl.ds(r0, cpt)])

  return k(gt, pk1)


def _dinvr(deg_parts):
  def body(dp_ref, or_ref):
    deg = jnp.sum(dp_ref[...], axis=0) + 1.0
    or_ref[...] = lax.rsqrt(deg)[None, :]

  return pl.pallas_call(
      body,
      out_shape=jax.ShapeDtypeStruct((1, NP), jnp.float32),
  )(deg_parts)


def _pack_edges(src_r, dst_r):
  rows_in = E // 128
  rows_out = EP // 128
  bl = rows_out // 10

  def body(s_ref, d_ref, o_ref):
    i = pl.program_id(0)
    row = lax.broadcasted_iota(jnp.int32, (bl, 128), 0) + i * bl
    pk = jnp.bitwise_or(s_ref[...], jnp.left_shift(d_ref[...], 16))
    o_ref[...] = jnp.where(row < rows_in, pk, DUMMY << 16)

  return pl.pallas_call(
      body,
      grid=(10,),
      in_specs=[
          pl.BlockSpec((bl, 128), lambda i: (i, 0)),
          pl.BlockSpec((bl, 128), lambda i: (i, 0)),
      ],
      out_specs=pl.BlockSpec((bl, 128), lambda i: (i, 0)),
      out_shape=jax.ShapeDtypeStruct((rows_out, 128), jnp.int32),
  )(src_r, dst_r)


def _tc_scale_matmul_t(xt, wt, dinvr):
  dout, din = wt.shape

  def body(x_ref, w_ref, dv_ref, o_ref):
    h = jnp.dot(w_ref[...], x_ref[...], preferred_element_type=jnp.float32)
    o_ref[...] = h * dv_ref[...]

  return pl.pallas_call(
      body,
      grid=(8,),
      in_specs=[
          pl.BlockSpec((din, 1280), lambda i: (0, i)),
          pl.BlockSpec((dout, din), lambda i: (0, 0)),
          pl.BlockSpec((1, 1280), lambda i: (0, i)),
      ],
      out_specs=pl.BlockSpec((dout, 1280), lambda i: (0, i)),
      out_shape=jax.ShapeDtypeStruct((dout, NP), jnp.float32),
  )(xt, wt, dinvr)


def _tc_combine_relu_matmul_t(st, gt, dinvr, bc, wt):
  dout, din = wt.shape

  def body(st_ref, gt_ref, dv_ref, b_ref, w_ref, o_ref):
    a = (st_ref[...] + gt_ref[...]) * dv_ref[...] + b_ref[...]
    r = jnp.maximum(a, 0.0)
    h = jnp.dot(w_ref[...], r, preferred_element_type=jnp.float32)
    o_ref[...] = h * dv_ref[...]

  return pl.pallas_call(
      body,
      grid=(4,),
      in_specs=[
          pl.BlockSpec((din, 2560), lambda i: (0, i)),
          pl.BlockSpec((din, 2560), lambda i: (0, i)),
          pl.BlockSpec((1, 2560), lambda i: (0, i)),
          pl.BlockSpec((din, 1), lambda i: (0, 0)),
          pl.BlockSpec((dout, din), lambda i: (0, 0)),
      ],
      out_specs=pl.BlockSpec((dout, 2560), lambda i: (0, i)),
      out_shape=jax.ShapeDtypeStruct((dout, NP), jnp.float32),
  )(st, gt, dinvr, bc, wt)


def _tc_combine_logsoftmax_t(st, gt, dinvr, bc):
  d = gt.shape[0]

  def body(st_ref, gt_ref, dv_ref, b_ref, o_ref):
    o = (st_ref[...] + gt_ref[...]) * dv_ref[...] + b_ref[...]
    m = jnp.max(o, axis=0, keepdims=True)
    e = jnp.exp(o - m)
    lse = jnp.log(jnp.sum(e, axis=0, keepdims=True))
    o_ref[...] = jnp.transpose(o - m - lse)

  return pl.pallas_call(
      body,
      grid=(4,),
      in_specs=[
          pl.BlockSpec((d, 2560), lambda i: (0, i)),
          pl.BlockSpec((d, 2560), lambda i: (0, i)),
          pl.BlockSpec((1, 2560), lambda i: (0, i)),
          pl.BlockSpec((d, 1), lambda i: (0, 0)),
      ],
      out_specs=pl.BlockSpec((2560, d), lambda i: (i, 0)),
      out_shape=jax.ShapeDtypeStruct((N, d), jnp.float32),
  )(st, gt, dinvr, bc)


def kernel(x, edge_index, W1, b1, W2, b2):
  src_r = edge_index[0].astype(jnp.int32).reshape(E // 128, 128)
  dst_r = edge_index[1].astype(jnp.int32).reshape(E // 128, 128)
  pk2 = _pack_edges(src_r, dst_r)
  pk1 = pk2.reshape(EP)
  pkh = pk2.reshape(NTILES, EP // NTILES)

  dparts = _deg_parts(pkh)
  dinvr = _dinvr(dparts)

  g1t = _tc_scale_matmul_t(x.T, W1.T, dinvr)
  s1t = _edge_scatter_t(g1t, pk1)
  g2t = _tc_combine_relu_matmul_t(s1t, g1t, dinvr, b1.reshape(-1, 1), W2.T)
  s2t = _edge_scatter_t(g2t, pk1)
  return _tc_combine_logsoftmax_t(s2t, g2t, dinvr, b2.reshape(-1, 1))


# speedup vs baseline: 1.1659x; 1.0416x over previous
import functools

import jax
import jax.numpy as jnp
from jax import lax
from jax.experimental import pallas as pl
from jax.experimental.pallas import tpu as pltpu
from jax.experimental.pallas import tpu_sc as plsc

N = 10000
E = 320000
NP = 10240
NTILES = 32
EP = 327680
PB = 5120
NPH = EP // PB
DUMMY = N

_mesh = plsc.VectorSubcoreMesh(core_axis_name="c", subcore_axis_name="s")
_params = pltpu.CompilerParams(needs_layout_passes=False)


def _deg_parts(pk1):
  ept = EP // NTILES

  @functools.partial(
      pl.kernel,
      out_type=jax.ShapeDtypeStruct((NTILES, NP), jnp.float32),
      mesh=_mesh,
      compiler_params=_params,
      scratch_types=[
          pltpu.VMEM((ept,), jnp.int32),
          pltpu.VMEM((NP,), jnp.float32),
      ],
  )
  def k(pk_hbm, out_hbm, pkv, hist):
    c = lax.axis_index("c")
    s = lax.axis_index("s")
    wid = c * 16 + s
    pltpu.sync_copy(pk_hbm.at[pl.ds(wid * ept, ept)], pkv)
    zeros = jnp.zeros((16,), jnp.float32)
    ones = jnp.ones((16,), jnp.float32)

    def zbody(i, carry):
      hist[pl.ds(i * 16, 16)] = zeros
      return carry

    lax.fori_loop(0, NP // 16, zbody, 0)

    @plsc.parallel_loop(0, ept // 16, unroll=8)
    def body(i):
      idx = lax.shift_right_logical(pkv[pl.ds(i * 16, 16)], 16)
      plsc.addupdate_scatter(hist, [idx], ones)
    pltpu.sync_copy(hist, out_hbm.at[wid])

  return k(pk1)


def _edge_scatter_t(gt, pk1):
  d = gt.shape[0]
  cpt = d // NTILES

  @functools.partial(
      pl.kernel,
      out_type=jax.ShapeDtypeStruct((d, NP), jnp.float32),
      mesh=_mesh,
      compiler_params=_params,
      scratch_types=[
          pltpu.VMEM((PB,), jnp.int32),
          pltpu.VMEM((PB,), jnp.int32),
          pltpu.VMEM((cpt, NP), jnp.float32),
          pltpu.VMEM((cpt, NP), jnp.float32),
          pltpu.SemaphoreType.DMA,
          pltpu.SemaphoreType.DMA,
      ],
  )
  def k(gt_hbm, pk_hbm, out_hbm, pkA, pkB, slab, acc, semA, semB):
    c = lax.axis_index("c")
    s = lax.axis_index("s")
    r0 = (c * 16 + s) * cpt
    pltpu.sync_copy(gt_hbm.at[pl.ds(r0, cpt)], slab)

    zeros = jnp.zeros((16,), jnp.float32)

    def zbody(i, carry):
      for cc in range(cpt):
        acc[cc, pl.ds(i * 16, 16)] = zeros
      return carry

    lax.fori_loop(0, NP // 16, zbody, 0)

    def start(ph, pv, sem):
      pltpu.async_copy(pk_hbm.at[pl.ds(ph * PB, PB)], pv, sem)

    def wait(pv, sem):
      pltpu.make_async_copy(pk_hbm.at[pl.ds(0, PB)], pv, sem).wait()

    mask = jnp.full((16,), 0xFFFF, jnp.int32)

    def process(pv):
      cvs = [jnp.full((16,), cc, jnp.int32) for cc in range(cpt)]

      @plsc.parallel_loop(0, PB // 16, unroll=4)
      def ibody(i):
        pk16 = pv[pl.ds(i * 16, 16)]
        s16 = jnp.bitwise_and(pk16, mask)
        d16 = lax.shift_right_logical(pk16, 16)
        for cc in range(cpt):
          val = plsc.load_gather(slab, [cvs[cc], s16])
          plsc.addupdate_scatter(acc, [cvs[cc], d16], val)

    start(0, pkA, semA)
    start(1, pkB, semB)

    def phases(i, carry):
      phA = 2 * i
      phB = 2 * i + 1
      wait(pkA, semA)
      process(pkA)
      start(jnp.minimum(phA + 2, NPH - 2), pkA, semA)
      wait(pkB, semB)
      process(pkB)
      start(jnp.minimum(phB + 2, NPH - 1), pkB, semB)
      return carry

    lax.fori_loop(0, NPH // 2, phases, 0)
    wait(pkA, semA)
    wait(pkB, semB)

    pltpu.sync_copy(acc, out_hbm.at[pl.ds(r0, cpt)])

  return k(gt, pk1)


def _dinvr(deg_parts):
  def body(dp_ref, or_ref):
    deg = jnp.sum(dp_ref[...], axis=0) + 1.0
    or_ref[...] = lax.rsqrt(deg)[None, :]

  return pl.pallas_call(
      body,
      out_shape=jax.ShapeDtypeStruct((1, NP), jnp.float32),
  )(deg_parts)


def _pack_edges(edge_index):
  bl = EP // 10

  def body(e_ref, o_ref):
    i = pl.program_id(0)
    gid = lax.broadcasted_iota(jnp.int32, (bl,), 0) + i * bl
    pk = jnp.bitwise_or(e_ref[0], jnp.left_shift(e_ref[1], 16))
    o_ref[...] = jnp.where(gid < E, pk, DUMMY << 16)

  return pl.pallas_call(
      body,
      grid=(10,),
      in_specs=[pl.BlockSpec((2, bl), lambda i: (0, i))],
      out_specs=pl.BlockSpec((bl,), lambda i: (i,)),
      out_shape=jax.ShapeDtypeStruct((EP,), jnp.int32),
  )(edge_index)


def _tc_scale_matmul_t(xt, wt, dinvr):
  dout, din = wt.shape

  def body(x_ref, w_ref, dv_ref, o_ref):
    h = jnp.dot(w_ref[...], x_ref[...], preferred_element_type=jnp.float32)
    o_ref[...] = h * dv_ref[...]

  return pl.pallas_call(
      body,
      grid=(8,),
      in_specs=[
          pl.BlockSpec((din, 1280), lambda i: (0, i)),
          pl.BlockSpec((dout, din), lambda i: (0, 0)),
          pl.BlockSpec((1, 1280), lambda i: (0, i)),
      ],
      out_specs=pl.BlockSpec((dout, 1280), lambda i: (0, i)),
      out_shape=jax.ShapeDtypeStruct((dout, NP), jnp.float32),
  )(xt, wt, dinvr)


def _tc_combine_relu_matmul_t(st, gt, dinvr, bc, wt):
  dout, din = wt.shape

  def body(st_ref, gt_ref, dv_ref, b_ref, w_ref, o_ref):
    a = (st_ref[...] + gt_ref[...]) * dv_ref[...] + b_ref[...]
    r = jnp.maximum(a, 0.0)
    h = jnp.dot(w_ref[...], r, preferred_element_type=jnp.float32)
    o_ref[...] = h * dv_ref[...]

  return pl.pallas_call(
      body,
      grid=(4,),
      in_specs=[
          pl.BlockSpec((din, 2560), lambda i: (0, i)),
          pl.BlockSpec((din, 2560), lambda i: (0, i)),
          pl.BlockSpec((1, 2560), lambda i: (0, i)),
          pl.BlockSpec((din, 1), lambda i: (0, 0)),
          pl.BlockSpec((dout, din), lambda i: (0, 0)),
      ],
      out_specs=pl.BlockSpec((dout, 2560), lambda i: (0, i)),
      out_shape=jax.ShapeDtypeStruct((dout, NP), jnp.float32),
  )(st, gt, dinvr, bc, wt)


def _tc_combine_logsoftmax_t(st, gt, dinvr, bc):
  d = gt.shape[0]

  def body(st_ref, gt_ref, dv_ref, b_ref, o_ref):
    o = (st_ref[...] + gt_ref[...]) * dv_ref[...] + b_ref[...]
    m = jnp.max(o, axis=0, keepdims=True)
    e = jnp.exp(o - m)
    lse = jnp.log(jnp.sum(e, axis=0, keepdims=True))
    o_ref[...] = jnp.transpose(o - m - lse)

  return pl.pallas_call(
      body,
      grid=(4,),
      in_specs=[
          pl.BlockSpec((d, 2560), lambda i: (0, i)),
          pl.BlockSpec((d, 2560), lambda i: (0, i)),
          pl.BlockSpec((1, 2560), lambda i: (0, i)),
          pl.BlockSpec((d, 1), lambda i: (0, 0)),
      ],
      out_specs=pl.BlockSpec((2560, d), lambda i: (i, 0)),
      out_shape=jax.ShapeDtypeStruct((N, d), jnp.float32),
  )(st, gt, dinvr, bc)


def kernel(x, edge_index, W1, b1, W2, b2):
  pk1 = _pack_edges(edge_index.astype(jnp.int32))

  dparts = _deg_parts(pk1)
  dinvr = _dinvr(dparts)

  g1t = _tc_scale_matmul_t(x.T, W1.T, dinvr)
  s1t = _edge_scatter_t(g1t, pk1)
  g2t = _tc_combine_relu_matmul_t(s1t, g1t, dinvr, b1.reshape(-1, 1), W2.T)
  s2t = _edge_scatter_t(g2t, pk1)
  return _tc_combine_logsoftmax_t(s2t, g2t, dinvr, b2.reshape(-1, 1))
